# baseline (speedup 1.0000x reference)
import dataclasses
import functools

import jax
import jax.numpy as jnp
from jax import lax
from jax.experimental import pallas as pl
from jax.experimental.pallas import tpu as pltpu
from jax.experimental.pallas import tpu_sc as plsc

NC = 2
NS = 16
NW = NC * NS
LANES = 16
NCLS = 3
CPAD = 8
VOCAB_PAD = 1024
UNROLL = 4


def _tw_body(table_ref, w_ref, b_ref, out_ref, *, inv_len):
    tw = lax.dot_general(
        w_ref[...], table_ref[...],
        (((1,), (1,)), ((), ())),
        preferred_element_type=jnp.float32,
    )
    out_ref[...] = (tw + b_ref[...]) * inv_len


def _compute_tw(table, W, b, seq_len):
    vocab = table.shape[0]
    tablep = jnp.pad(table, ((0, VOCAB_PAD - vocab), (0, 0)))
    wp = jnp.pad(W, ((0, CPAD - NCLS), (0, 0)))
    bp = jnp.pad(b, (0, CPAD - NCLS)).reshape(CPAD, 1)
    return pl.pallas_call(
        functools.partial(_tw_body, inv_len=1.0 / seq_len),
        out_shape=jax.ShapeDtypeStruct((CPAD, VOCAB_PAD), jnp.float32),
    )(tablep, wp, bp)


def _make_sc_forward(batch, seq_len):
    b_per_w = batch // NW
    gdma = 64
    n_blocks = b_per_w // gdma
    n_sub = gdma // LANES
    n_steps = seq_len // UNROLL

    mesh = plsc.VectorSubcoreMesh(core_axis_name="c", subcore_axis_name="s")
    cp = pltpu.CompilerParams()
    if "needs_layout_passes" in pltpu.CompilerParams.__dataclass_fields__:
        cp = dataclasses.replace(cp, needs_layout_passes=False)

    @functools.partial(
        pl.kernel,
        mesh=mesh,
        compiler_params=cp,
        out_type=jax.ShapeDtypeStruct((batch * NCLS,), jnp.float32),
        scratch_types=[
            pltpu.VMEM((VOCAB_PAD,), jnp.float32),
            pltpu.VMEM((VOCAB_PAD,), jnp.float32),
            pltpu.VMEM((VOCAB_PAD,), jnp.float32),
            pltpu.VMEM((gdma, seq_len), jnp.int32),
            pltpu.VMEM((b_per_w * NCLS,), jnp.float32),
        ],
    )
    def sc_forward(tw_hbm, x_hbm, out_hbm, tw0, tw1, tw2, xv, ov):
        wid = lax.axis_index("s") * NC + lax.axis_index("c")
        row_base = wid * b_per_w
        pltpu.sync_copy(tw_hbm.at[0], tw0)
        pltpu.sync_copy(tw_hbm.at[1], tw1)
        pltpu.sync_copy(tw_hbm.at[2], tw2)
        lanes = lax.iota(jnp.int32, LANES)

        @pl.loop(0, n_blocks)
        def _block(blk):
            pltpu.sync_copy(x_hbm.at[pl.ds(row_base + blk * gdma, gdma), :], xv)
            for sub in range(n_sub):
                rows = sub * LANES + lanes

                def body(i, accs):
                    a0, a1, a2 = accs
                    for u in range(UNROLL):
                        lpos = jnp.full((LANES,), 0, jnp.int32) + (i * UNROLL + u)
                        idx = plsc.load_gather(xv, [rows, lpos])
                        a0 = a0 + plsc.load_gather(tw0, [idx])
                        a1 = a1 + plsc.load_gather(tw1, [idx])
                        a2 = a2 + plsc.load_gather(tw2, [idx])
                    return a0, a1, a2

                z = jnp.zeros((LANES,), jnp.float32)
                a0, a1, a2 = lax.fori_loop(0, n_steps, body, (z, z, z))

                m = jnp.maximum(jnp.maximum(a0, a1), a2)
                e0 = jnp.exp(a0 - m)
                e1 = jnp.exp(a1 - m)
                e2 = jnp.exp(a2 - m)
                r = 1.0 / (e0 + e1 + e2)
                oaddr = (blk * gdma + rows) * NCLS
                plsc.store_scatter(ov, [oaddr], e0 * r)
                plsc.store_scatter(ov, [oaddr + 1], e1 * r)
                plsc.store_scatter(ov, [oaddr + 2], e2 * r)

        pltpu.sync_copy(ov, out_hbm.at[pl.ds(wid * b_per_w * NCLS, b_per_w * NCLS)])

    return sc_forward


def kernel(sentenceX, table, W, b):
    batch, seq_len = sentenceX.shape
    tw = _compute_tw(table, W, b, seq_len)
    out_flat = _make_sc_forward(batch, seq_len)(tw, sentenceX.astype(jnp.int32))
    return out_flat.reshape(batch, NCLS)


# speedup vs baseline: 1.2795x; 1.2795x over previous
import dataclasses
import functools

import jax
import jax.numpy as jnp
from jax import lax
from jax.experimental import pallas as pl
from jax.experimental.pallas import tpu as pltpu
from jax.experimental.pallas import tpu_sc as plsc

NC = 2
NS = 16
NW = NC * NS
LANES = 16
NCLS = 3
CPAD = 8
VOCAB_PAD = 1024
UNROLL = 8


def _tw_body(table_ref, w_ref, b_ref, out_ref, *, inv_len):
    tw = lax.dot_general(
        w_ref[...], table_ref[...],
        (((1,), (1,)), ((), ())),
        preferred_element_type=jnp.float32,
    )
    out_ref[...] = (tw + b_ref[...]) * inv_len


def _compute_tw(table, W, b, seq_len):
    wd = jnp.stack([W[0] - W[2], W[1] - W[2]])
    bd = jnp.stack([b[0] - b[2], b[1] - b[2]])
    vocab = table.shape[0]
    tablep = jnp.pad(table, ((0, VOCAB_PAD - vocab), (0, 0)))
    wp = jnp.pad(wd, ((0, CPAD - 2), (0, 0)))
    bp = jnp.pad(bd, (0, CPAD - 2)).reshape(CPAD, 1)
    return pl.pallas_call(
        functools.partial(_tw_body, inv_len=1.0 / seq_len),
        out_shape=jax.ShapeDtypeStruct((CPAD, VOCAB_PAD), jnp.float32),
    )(tablep, wp, bp)


def _make_sc_forward(batch, seq_len):
    b_per_w = batch // NW
    n_groups = b_per_w // LANES
    chunk = b_per_w * seq_len
    n_steps = seq_len // UNROLL

    mesh = plsc.VectorSubcoreMesh(core_axis_name="c", subcore_axis_name="s")
    cp = pltpu.CompilerParams()
    if "needs_layout_passes" in pltpu.CompilerParams.__dataclass_fields__:
        cp = dataclasses.replace(cp, needs_layout_passes=False)

    @functools.partial(
        pl.kernel,
        mesh=mesh,
        compiler_params=cp,
        out_type=jax.ShapeDtypeStruct((batch * NCLS,), jnp.float32),
        scratch_types=[
            pltpu.VMEM((VOCAB_PAD,), jnp.float32),
            pltpu.VMEM((VOCAB_PAD,), jnp.float32),
            pltpu.VMEM((chunk,), jnp.int32),
            pltpu.VMEM((b_per_w * NCLS,), jnp.float32),
        ],
    )
    def sc_forward(tw_hbm, x_hbm, out_hbm, d0, d1, xv, ov):
        wid = lax.axis_index("s") * NC + lax.axis_index("c")
        base = wid * chunk
        pltpu.sync_copy(tw_hbm.at[0], d0)
        pltpu.sync_copy(tw_hbm.at[1], d1)
        pltpu.sync_copy(x_hbm.at[pl.ds(base, chunk)], xv)
        lanes = lax.iota(jnp.int32, LANES)

        @pl.loop(0, n_groups)
        def _group(g):
            rows = g * LANES + lanes
            addr0 = rows * seq_len

            def body(i, accs):
                a0, a1 = accs
                idxs = [
                    plsc.load_gather(xv, [addr0 + (i * UNROLL + u)])
                    for u in range(UNROLL)
                ]
                for idx in idxs:
                    a0 = a0 + plsc.load_gather(d0, [idx])
                    a1 = a1 + plsc.load_gather(d1, [idx])
                return a0, a1

            z = jnp.zeros((LANES,), jnp.float32)
            a0, a1 = lax.fori_loop(0, n_steps, body, (z, z))

            m = jnp.maximum(jnp.maximum(a0, a1), 0.0)
            e0 = jnp.exp(a0 - m)
            e1 = jnp.exp(a1 - m)
            e2 = jnp.exp(-m)
            r = 1.0 / (e0 + e1 + e2)
            oaddr = rows * NCLS
            plsc.store_scatter(ov, [oaddr], e0 * r)
            plsc.store_scatter(ov, [oaddr + 1], e1 * r)
            plsc.store_scatter(ov, [oaddr + 2], e2 * r)

        pltpu.sync_copy(ov, out_hbm.at[pl.ds(wid * b_per_w * NCLS, b_per_w * NCLS)])

    return sc_forward


def kernel(sentenceX, table, W, b):
    batch, seq_len = sentenceX.shape
    tw = _compute_tw(table, W, b, seq_len)
    x_flat = sentenceX.astype(jnp.int32).reshape(-1)
    out_flat = _make_sc_forward(batch, seq_len)(tw, x_flat)
    return out_flat.reshape(batch, NCLS)


# speedup vs baseline: 1.4808x; 1.1574x over previous
import dataclasses
import functools

import jax
import jax.numpy as jnp
from jax import lax
from jax.experimental import pallas as pl
from jax.experimental.pallas import tpu as pltpu
from jax.experimental.pallas import tpu_sc as plsc

NC = 2
NS = 16
NW = NC * NS
LANES = 16
NCLS = 3
CPAD = 8
VOCAB_PAD = 1024
UNROLL = 4


def _tw_body(table_ref, w_ref, b_ref, out_ref, *, inv_len):
    tw = lax.dot_general(
        w_ref[...], table_ref[...],
        (((1,), (1,)), ((), ())),
        preferred_element_type=jnp.float32,
    )
    out_ref[...] = (tw + b_ref[...]) * inv_len


def _compute_tw(table, W, b, seq_len):
    wd = jnp.stack([W[0] - W[2], W[1] - W[2]])
    bd = jnp.stack([b[0] - b[2], b[1] - b[2]])
    vocab = table.shape[0]
    tablep = jnp.pad(table, ((0, VOCAB_PAD - vocab), (0, 0)))
    wp = jnp.pad(wd, ((0, CPAD - 2), (0, 0)))
    bp = jnp.pad(bd, (0, CPAD - 2)).reshape(CPAD, 1)
    return pl.pallas_call(
        functools.partial(_tw_body, inv_len=1.0 / seq_len),
        out_shape=jax.ShapeDtypeStruct((CPAD, VOCAB_PAD), jnp.float32),
    )(tablep, wp, bp)


def _make_sc_forward(batch, seq_pk):
    b_per_w = batch // NW
    n_groups = b_per_w // LANES
    chunk = b_per_w * seq_pk
    n_steps = seq_pk // UNROLL

    mesh = plsc.VectorSubcoreMesh(core_axis_name="c", subcore_axis_name="s")
    cp = pltpu.CompilerParams()
    if "needs_layout_passes" in pltpu.CompilerParams.__dataclass_fields__:
        cp = dataclasses.replace(cp, needs_layout_passes=False)

    @functools.partial(
        pl.kernel,
        mesh=mesh,
        compiler_params=cp,
        out_type=jax.ShapeDtypeStruct((batch * NCLS,), jnp.float32),
        scratch_types=[
            pltpu.VMEM((VOCAB_PAD,), jnp.float32),
            pltpu.VMEM((VOCAB_PAD,), jnp.float32),
            pltpu.VMEM((chunk,), jnp.int32),
            pltpu.VMEM((b_per_w * NCLS,), jnp.float32),
        ],
    )
    def sc_forward(tw_hbm, x_hbm, out_hbm, d0, d1, xv, ov):
        wid = lax.axis_index("s") * NC + lax.axis_index("c")
        base = wid * chunk
        pltpu.sync_copy(tw_hbm.at[0], d0)
        pltpu.sync_copy(tw_hbm.at[1], d1)
        pltpu.sync_copy(x_hbm.at[pl.ds(base, chunk)], xv)
        lanes = lax.iota(jnp.int32, LANES)

        @pl.loop(0, n_groups)
        def _group(g):
            rows = g * LANES + lanes
            addr0 = rows * seq_pk

            def body(i, accs):
                a0, a1 = accs
                packs = [
                    plsc.load_gather(xv, [addr0 + (i * UNROLL + u)])
                    for u in range(UNROLL)
                ]
                idxs = []
                for v in packs:
                    idxs.append(jnp.bitwise_and(v, 0xFFFF))
                    idxs.append(lax.shift_right_logical(v, 16))
                for idx in idxs:
                    a0 = a0 + plsc.load_gather(d0, [idx])
                    a1 = a1 + plsc.load_gather(d1, [idx])
                return a0, a1

            z = jnp.zeros((LANES,), jnp.float32)
            a0, a1 = lax.fori_loop(0, n_steps, body, (z, z))

            m = jnp.maximum(jnp.maximum(a0, a1), 0.0)
            e0 = jnp.exp(a0 - m)
            e1 = jnp.exp(a1 - m)
            e2 = jnp.exp(-m)
            r = 1.0 / (e0 + e1 + e2)
            oaddr = rows * NCLS
            plsc.store_scatter(ov, [oaddr], e0 * r)
            plsc.store_scatter(ov, [oaddr + 1], e1 * r)
            plsc.store_scatter(ov, [oaddr + 2], e2 * r)

        pltpu.sync_copy(ov, out_hbm.at[pl.ds(wid * b_per_w * NCLS, b_per_w * NCLS)])

    return sc_forward


def kernel(sentenceX, table, W, b):
    batch, seq_len = sentenceX.shape
    half = seq_len // 2
    tw = _compute_tw(table, W, b, seq_len)
    x = sentenceX.astype(jnp.int32)
    x_pk = jnp.bitwise_or(x[:, :half], x[:, half:] << 16).reshape(-1)
    out_flat = _make_sc_forward(batch, half)(tw, x_pk)
    return out_flat.reshape(batch, NCLS)
